# baseline (speedup 1.0000x reference)
import functools

import jax
import jax.numpy as jnp
from jax import lax
from jax.experimental import pallas as pl
from jax.experimental.pallas import tpu as pltpu
from jax.experimental.pallas import tpu_sc as plsc

N_NODES = 10000
N_EDGES = 320000
D = 128

NC = 2
NS = 16
NW = NC * NS

NPAD = 10240
G = 64
EPAD = 327680
GROUPS_TOT = EPAD // G
GPT = GROUPS_TOT // NW
NB = 4
RPT = NPAD // NS

_mesh = plsc.VectorSubcoreMesh(core_axis_name="c", subcore_axis_name="s")


CH = 16
GPT0 = 160
GPT1 = 160


def _sc_agg_body(with_cnt, *refs):
    if with_cnt:
        (h_hbm, src_hbm, dst_hbm, z2d, z1d,
         agg0, agg1, cnt0, cnt1,
         sidx, didx, rows, ones, agg_sh, cnt_sh, gsem, ssem, csem, isem) = refs
    else:
        (h_hbm, src_hbm, dst_hbm, z2d,
         agg0, agg1,
         sidx, didx, rows, agg_sh, gsem, ssem, isem) = refs

    c = lax.axis_index("c")
    s = lax.axis_index("s")
    base = jnp.where(c == 0, s * GPT0, NS * GPT0 + s * GPT1)
    nch = jnp.where(c == 0, GPT0 // CH, GPT1 // CH)

    pltpu.sync_copy(z2d.at[pl.ds(s * RPT, RPT)], agg_sh.at[pl.ds(s * RPT, RPT)])
    if with_cnt:
        pltpu.sync_copy(z1d.at[pl.ds(s * RPT, RPT)], cnt_sh.at[pl.ds(s * RPT, RPT)])
        for k in range(G // 16):
            ones[pl.ds(k * 16, 16)] = jnp.full((16,), 1.0, jnp.float32)

    pltpu.sync_copy(src_hbm.at[pl.ds(base, CH)], sidx.at[0])
    pltpu.sync_copy(dst_hbm.at[pl.ds(base, CH)], didx.at[0])
    plsc.subcore_barrier()

    def chunk(ci, carry):
        buf = lax.rem(ci, 2)
        nbuf = lax.rem(ci + 1, 2)
        base_next = base + (ci + 1) * CH

        @pl.when(ci + 1 < nch)
        def _():
            pltpu.async_copy(src_hbm.at[pl.ds(base_next, CH)], sidx.at[nbuf], isem)
            pltpu.async_copy(dst_hbm.at[pl.ds(base_next, CH)], didx.at[nbuf], isem)

        for p in range(NB - 1):
            pltpu.async_copy(h_hbm.at[sidx.at[buf, p]], rows.at[p], gsem)
        for j in range(CH):
            b = j % NB
            pltpu.make_async_copy(
                h_hbm.at[sidx.at[buf, j]], rows.at[b], gsem).wait()
            if j + NB - 1 < CH:
                nb = (j + NB - 1) % NB
                if j >= 1:
                    pltpu.make_async_copy(
                        rows.at[nb], agg_sh.at[didx.at[buf, j]], ssem).wait()
                pltpu.async_copy(
                    h_hbm.at[sidx.at[buf, j + NB - 1]], rows.at[nb], gsem)
            pltpu.async_copy(rows.at[b], agg_sh.at[didx.at[buf, j]], ssem, add=True)
            if with_cnt:
                pltpu.async_copy(ones, cnt_sh.at[didx.at[buf, j]], csem, add=True)

        for k in range(NB):
            pltpu.make_async_copy(
                rows.at[k], agg_sh.at[didx.at[buf, 0]], ssem).wait()
        if with_cnt:
            for k in range(CH):
                pltpu.make_async_copy(
                    ones, cnt_sh.at[didx.at[buf, 0]], csem).wait()

        @pl.when(ci + 1 < nch)
        def _():
            pltpu.make_async_copy(
                src_hbm.at[pl.ds(base_next, CH)], sidx.at[nbuf], isem).wait()
            pltpu.make_async_copy(
                dst_hbm.at[pl.ds(base_next, CH)], didx.at[nbuf], isem).wait()
        return carry

    lax.fori_loop(0, nch, chunk, 0)
    plsc.subcore_barrier()

    sl = pl.ds(s * RPT, RPT)

    @pl.when(c == 0)
    def _():
        pltpu.sync_copy(agg_sh.at[sl], agg0.at[sl])
        if with_cnt:
            pltpu.sync_copy(cnt_sh.at[sl], cnt0.at[sl])

    @pl.when(c == 1)
    def _():
        pltpu.sync_copy(agg_sh.at[sl], agg1.at[sl])
        if with_cnt:
            pltpu.sync_copy(cnt_sh.at[sl], cnt1.at[sl])


_sc_agg_cnt = pl.kernel(
    functools.partial(_sc_agg_body, True),
    out_type=(
        jax.ShapeDtypeStruct((NPAD, D), jnp.float32),
        jax.ShapeDtypeStruct((NPAD, D), jnp.float32),
        jax.ShapeDtypeStruct((NPAD,), jnp.float32),
        jax.ShapeDtypeStruct((NPAD,), jnp.float32),
    ),
    mesh=_mesh,
    scratch_types=[
        pltpu.VMEM((2, CH, G), jnp.int32),
        pltpu.VMEM((2, CH, G), jnp.int32),
        pltpu.VMEM((NB, G, D), jnp.float32),
        pltpu.VMEM((G,), jnp.float32),
        pltpu.VMEM_SHARED((NPAD, D), jnp.float32),
        pltpu.VMEM_SHARED((NPAD,), jnp.float32),
        pltpu.SemaphoreType.DMA,
        pltpu.SemaphoreType.DMA,
        pltpu.SemaphoreType.DMA,
        pltpu.SemaphoreType.DMA,
    ],
)

_sc_agg = pl.kernel(
    functools.partial(_sc_agg_body, False),
    out_type=(
        jax.ShapeDtypeStruct((NPAD, D), jnp.float32),
        jax.ShapeDtypeStruct((NPAD, D), jnp.float32),
    ),
    mesh=_mesh,
    scratch_types=[
        pltpu.VMEM((2, CH, G), jnp.int32),
        pltpu.VMEM((2, CH, G), jnp.int32),
        pltpu.VMEM((NB, G, D), jnp.float32),
        pltpu.VMEM_SHARED((NPAD, D), jnp.float32),
        pltpu.SemaphoreType.DMA,
        pltpu.SemaphoreType.DMA,
        pltpu.SemaphoreType.DMA,
    ],
)


BLK = 2048


def _tc_layer_body(act, h_ref, a0_ref, a1_ref, c0_ref, c1_ref,
                   wl_ref, bl_ref, wr_ref, o_ref):
    cnt = c0_ref[...] + c1_ref[...]
    inv = 1.0 / jnp.clip(cnt, 1.0, None)
    mean = (a0_ref[...].astype(jnp.float32)
            + a1_ref[...].astype(jnp.float32)) * inv
    z = (jnp.dot(mean, wl_ref[...], preferred_element_type=jnp.float32)
         + bl_ref[...]
         + jnp.dot(h_ref[...], wr_ref[...], preferred_element_type=jnp.float32))
    if act == "relu":
        z = jnp.maximum(z, 0.0)
    else:
        m = jnp.max(z, axis=-1, keepdims=True)
        e = jnp.exp(z - m)
        z = z - m - jnp.log(jnp.sum(e, axis=-1, keepdims=True))
    o_ref[...] = z


def _tc_layer(h, a0, a1, c0, c1, wlT, bl, wrT, act):
    row_spec = pl.BlockSpec((BLK, D), lambda i: (i, 0))
    cnt_spec = pl.BlockSpec((BLK, 1), lambda i: (i, 0))
    full = pl.BlockSpec((D, D), lambda i: (0, 0))
    bspec = pl.BlockSpec((1, D), lambda i: (0, 0))
    return pl.pallas_call(
        functools.partial(_tc_layer_body, act),
        grid=(NPAD // BLK,),
        in_specs=[row_spec, row_spec, row_spec, cnt_spec, cnt_spec,
                  full, bspec, full],
        out_specs=row_spec,
        out_shape=jax.ShapeDtypeStruct((NPAD, D), jnp.float32),
    )(h, a0, a1, c0, c1, wlT, bl, wrT)


def kernel(x, edge_index, Wl1, bl1, Wr1, Wl2, bl2, Wr2, Wl3, bl3, Wr3):
    ei = edge_index.astype(jnp.int32)
    src = jnp.concatenate(
        [ei[0], jnp.zeros((EPAD - N_EDGES,), jnp.int32)]).reshape(GROUPS_TOT, G)
    dst = jnp.concatenate(
        [ei[1], jnp.full((EPAD - N_EDGES,), N_NODES, jnp.int32)]).reshape(GROUPS_TOT, G)
    h0 = jnp.pad(x, ((0, NPAD - N_NODES), (0, 0)))
    z2d = jnp.zeros((NPAD, D), jnp.float32)
    z1d = jnp.zeros((NPAD,), jnp.float32)

    a0, a1, c0, c1 = _sc_agg_cnt(h0, src, dst, z2d, z1d)
    c0 = c0[:, None]
    c1 = c1[:, None]
    h1 = _tc_layer(h0, a0, a1, c0, c1, Wl1.T, bl1[None], Wr1.T, "relu")
    a0, a1 = _sc_agg(h1, src, dst, z2d)
    h2 = _tc_layer(h1, a0, a1, c0, c1, Wl2.T, bl2[None], Wr2.T, "relu")
    a0, a1 = _sc_agg(h2, src, dst, z2d)
    out = _tc_layer(h2, a0, a1, c0, c1, Wl3.T, bl3[None], Wr3.T, "logsoftmax")
    return out[:N_NODES]


# speedup vs baseline: 1.0523x; 1.0523x over previous
import functools

import jax
import jax.numpy as jnp
from jax import lax
from jax.experimental import pallas as pl
from jax.experimental.pallas import tpu as pltpu
from jax.experimental.pallas import tpu_sc as plsc

N_NODES = 10000
N_EDGES = 320000
D = 128

NC = 2
NS = 16
NW = NC * NS

NPAD = 10240
G = 64
EPAD = 327680
GROUPS_TOT = EPAD // G
GPT = GROUPS_TOT // NW
NB = 4
RPT = NPAD // NS

_mesh = plsc.VectorSubcoreMesh(core_axis_name="c", subcore_axis_name="s")


CH = 16
GPT0 = 256
GPT1 = 64


def _sc_agg_body(with_cnt, *refs):
    if with_cnt:
        (h_hbm, src_hbm, dst_hbm, z2d, z1d,
         agg0, agg1, cnt0, cnt1,
         sidx, didx, rows, ones, agg_sh, cnt_sh, gsem, ssem, csem, isem) = refs
    else:
        (h_hbm, src_hbm, dst_hbm, z2d,
         agg0, agg1,
         sidx, didx, rows, agg_sh, gsem, ssem, isem) = refs

    c = lax.axis_index("c")
    s = lax.axis_index("s")
    base = jnp.where(c == 0, s * GPT0, NS * GPT0 + s * GPT1)
    nch = jnp.where(c == 0, GPT0 // CH, GPT1 // CH)

    pltpu.sync_copy(z2d.at[pl.ds(s * RPT, RPT)], agg_sh.at[pl.ds(s * RPT, RPT)])
    if with_cnt:
        pltpu.sync_copy(z1d.at[pl.ds(s * RPT, RPT)], cnt_sh.at[pl.ds(s * RPT, RPT)])
        for k in range(G // 16):
            ones[pl.ds(k * 16, 16)] = jnp.full((16,), 1.0, jnp.float32)

    pltpu.sync_copy(src_hbm.at[pl.ds(base, CH)], sidx.at[0])
    pltpu.sync_copy(dst_hbm.at[pl.ds(base, CH)], didx.at[0])
    plsc.subcore_barrier()

    def chunk(ci, carry):
        buf = lax.rem(ci, 2)
        nbuf = lax.rem(ci + 1, 2)
        base_next = base + (ci + 1) * CH

        @pl.when(ci + 1 < nch)
        def _():
            pltpu.async_copy(src_hbm.at[pl.ds(base_next, CH)], sidx.at[nbuf], isem)
            pltpu.async_copy(dst_hbm.at[pl.ds(base_next, CH)], didx.at[nbuf], isem)

        for p in range(NB - 1):
            pltpu.async_copy(h_hbm.at[sidx.at[buf, p]], rows.at[p], gsem)
        for j in range(CH):
            b = j % NB
            pltpu.make_async_copy(
                h_hbm.at[sidx.at[buf, j]], rows.at[b], gsem).wait()
            if j + NB - 1 < CH:
                nb = (j + NB - 1) % NB
                if j >= 1:
                    pltpu.make_async_copy(
                        rows.at[nb], agg_sh.at[didx.at[buf, j]], ssem).wait()
                pltpu.async_copy(
                    h_hbm.at[sidx.at[buf, j + NB - 1]], rows.at[nb], gsem)
            pltpu.async_copy(rows.at[b], agg_sh.at[didx.at[buf, j]], ssem, add=True)
            if with_cnt:
                pltpu.async_copy(ones, cnt_sh.at[didx.at[buf, j]], csem, add=True)

        for k in range(NB):
            pltpu.make_async_copy(
                rows.at[k], agg_sh.at[didx.at[buf, 0]], ssem).wait()
        if with_cnt:
            for k in range(CH):
                pltpu.make_async_copy(
                    ones, cnt_sh.at[didx.at[buf, 0]], csem).wait()

        @pl.when(ci + 1 < nch)
        def _():
            pltpu.make_async_copy(
                src_hbm.at[pl.ds(base_next, CH)], sidx.at[nbuf], isem).wait()
            pltpu.make_async_copy(
                dst_hbm.at[pl.ds(base_next, CH)], didx.at[nbuf], isem).wait()
        return carry

    lax.fori_loop(0, nch, chunk, 0)
    plsc.subcore_barrier()

    sl = pl.ds(s * RPT, RPT)

    @pl.when(c == 0)
    def _():
        pltpu.sync_copy(agg_sh.at[sl], agg0.at[sl])
        if with_cnt:
            pltpu.sync_copy(cnt_sh.at[sl], cnt0.at[sl])

    @pl.when(c == 1)
    def _():
        pltpu.sync_copy(agg_sh.at[sl], agg1.at[sl])
        if with_cnt:
            pltpu.sync_copy(cnt_sh.at[sl], cnt1.at[sl])


_sc_agg_cnt = pl.kernel(
    functools.partial(_sc_agg_body, True),
    out_type=(
        jax.ShapeDtypeStruct((NPAD, D), jnp.float32),
        jax.ShapeDtypeStruct((NPAD, D), jnp.float32),
        jax.ShapeDtypeStruct((NPAD,), jnp.float32),
        jax.ShapeDtypeStruct((NPAD,), jnp.float32),
    ),
    mesh=_mesh,
    scratch_types=[
        pltpu.VMEM((2, CH, G), jnp.int32),
        pltpu.VMEM((2, CH, G), jnp.int32),
        pltpu.VMEM((NB, G, D), jnp.float32),
        pltpu.VMEM((G,), jnp.float32),
        pltpu.VMEM_SHARED((NPAD, D), jnp.float32),
        pltpu.VMEM_SHARED((NPAD,), jnp.float32),
        pltpu.SemaphoreType.DMA,
        pltpu.SemaphoreType.DMA,
        pltpu.SemaphoreType.DMA,
        pltpu.SemaphoreType.DMA,
    ],
)

_sc_agg = pl.kernel(
    functools.partial(_sc_agg_body, False),
    out_type=(
        jax.ShapeDtypeStruct((NPAD, D), jnp.float32),
        jax.ShapeDtypeStruct((NPAD, D), jnp.float32),
    ),
    mesh=_mesh,
    scratch_types=[
        pltpu.VMEM((2, CH, G), jnp.int32),
        pltpu.VMEM((2, CH, G), jnp.int32),
        pltpu.VMEM((NB, G, D), jnp.float32),
        pltpu.VMEM_SHARED((NPAD, D), jnp.float32),
        pltpu.SemaphoreType.DMA,
        pltpu.SemaphoreType.DMA,
        pltpu.SemaphoreType.DMA,
    ],
)


BLK = 2048


def _tc_layer_body(act, h_ref, a0_ref, a1_ref, c0_ref, c1_ref,
                   wl_ref, bl_ref, wr_ref, o_ref):
    cnt = c0_ref[...] + c1_ref[...]
    inv = 1.0 / jnp.clip(cnt, 1.0, None)
    mean = (a0_ref[...].astype(jnp.float32)
            + a1_ref[...].astype(jnp.float32)) * inv
    z = (jnp.dot(mean, wl_ref[...], preferred_element_type=jnp.float32)
         + bl_ref[...]
         + jnp.dot(h_ref[...], wr_ref[...], preferred_element_type=jnp.float32))
    if act == "relu":
        z = jnp.maximum(z, 0.0)
    else:
        m = jnp.max(z, axis=-1, keepdims=True)
        e = jnp.exp(z - m)
        z = z - m - jnp.log(jnp.sum(e, axis=-1, keepdims=True))
    o_ref[...] = z


def _tc_layer(h, a0, a1, c0, c1, wlT, bl, wrT, act):
    row_spec = pl.BlockSpec((BLK, D), lambda i: (i, 0))
    cnt_spec = pl.BlockSpec((BLK, 1), lambda i: (i, 0))
    full = pl.BlockSpec((D, D), lambda i: (0, 0))
    bspec = pl.BlockSpec((1, D), lambda i: (0, 0))
    return pl.pallas_call(
        functools.partial(_tc_layer_body, act),
        grid=(NPAD // BLK,),
        in_specs=[row_spec, row_spec, row_spec, cnt_spec, cnt_spec,
                  full, bspec, full],
        out_specs=row_spec,
        out_shape=jax.ShapeDtypeStruct((NPAD, D), jnp.float32),
    )(h, a0, a1, c0, c1, wlT, bl, wrT)


def kernel(x, edge_index, Wl1, bl1, Wr1, Wl2, bl2, Wr2, Wl3, bl3, Wr3):
    ei = edge_index.astype(jnp.int32)
    src = jnp.concatenate(
        [ei[0], jnp.zeros((EPAD - N_EDGES,), jnp.int32)]).reshape(GROUPS_TOT, G)
    dst = jnp.concatenate(
        [ei[1], jnp.full((EPAD - N_EDGES,), N_NODES, jnp.int32)]).reshape(GROUPS_TOT, G)
    h0 = jnp.pad(x, ((0, NPAD - N_NODES), (0, 0)))
    z2d = jnp.zeros((NPAD, D), jnp.float32)
    z1d = jnp.zeros((NPAD,), jnp.float32)

    a0, a1, c0, c1 = _sc_agg_cnt(h0, src, dst, z2d, z1d)
    c0 = c0[:, None]
    c1 = c1[:, None]
    h1 = _tc_layer(h0, a0, a1, c0, c1, Wl1.T, bl1[None], Wr1.T, "relu")
    a0, a1 = _sc_agg(h1, src, dst, z2d)
    h2 = _tc_layer(h1, a0, a1, c0, c1, Wl2.T, bl2[None], Wr2.T, "relu")
    a0, a1 = _sc_agg(h2, src, dst, z2d)
    out = _tc_layer(h2, a0, a1, c0, c1, Wl3.T, bl3[None], Wr3.T, "logsoftmax")
    return out[:N_NODES]


# speedup vs baseline: 1.1982x; 1.1387x over previous
import functools

import jax
import jax.numpy as jnp
from jax import lax
from jax.experimental import pallas as pl
from jax.experimental.pallas import tpu as pltpu
from jax.experimental.pallas import tpu_sc as plsc

N_NODES = 10000
N_EDGES = 320000
D = 128

NC = 2
NS = 16
NW = NC * NS

NPAD = 10240
G = 64
EPAD = 327680
GROUPS_TOT = EPAD // G
GPT = GROUPS_TOT // NW
NB = 4
RPT = NPAD // NS

_mesh = plsc.VectorSubcoreMesh(core_axis_name="c", subcore_axis_name="s")


CH = 16
GPT0 = 288
GPT1 = 32


def _sc_agg_body(with_cnt, *refs):
    if with_cnt:
        (h_hbm, src_hbm, dst_hbm, z2d, z1d,
         agg0, agg1, cnt0, cnt1,
         sidx, didx, rows, ones, agg_sh, cnt_sh, gsem, ssem, csem, isem) = refs
    else:
        (h_hbm, src_hbm, dst_hbm, z2d,
         agg0, agg1,
         sidx, didx, rows, agg_sh, gsem, ssem, isem) = refs

    c = lax.axis_index("c")
    s = lax.axis_index("s")
    base = jnp.where(c == 0, s * GPT0, NS * GPT0 + s * GPT1)
    nch = jnp.where(c == 0, GPT0 // CH, GPT1 // CH)

    pltpu.sync_copy(z2d.at[pl.ds(s * RPT, RPT)], agg_sh.at[pl.ds(s * RPT, RPT)])
    if with_cnt:
        pltpu.sync_copy(z1d.at[pl.ds(s * RPT, RPT)], cnt_sh.at[pl.ds(s * RPT, RPT)])
        for k in range(G // 16):
            ones[pl.ds(k * 16, 16)] = jnp.full((16,), 1.0, jnp.float32)

    pltpu.sync_copy(src_hbm.at[pl.ds(base, CH)], sidx.at[0])
    pltpu.sync_copy(dst_hbm.at[pl.ds(base, CH)], didx.at[0])
    plsc.subcore_barrier()

    def chunk(ci, carry):
        buf = lax.rem(ci, 2)
        nbuf = lax.rem(ci + 1, 2)
        base_next = base + (ci + 1) * CH

        @pl.when(ci + 1 < nch)
        def _():
            pltpu.async_copy(src_hbm.at[pl.ds(base_next, CH)], sidx.at[nbuf], isem)
            pltpu.async_copy(dst_hbm.at[pl.ds(base_next, CH)], didx.at[nbuf], isem)

        for p in range(NB - 1):
            pltpu.async_copy(h_hbm.at[sidx.at[buf, p]], rows.at[p], gsem)
        for j in range(CH):
            b = j % NB
            pltpu.make_async_copy(
                h_hbm.at[sidx.at[buf, j]], rows.at[b], gsem).wait()
            if j + NB - 1 < CH:
                nb = (j + NB - 1) % NB
                if j >= 1:
                    pltpu.make_async_copy(
                        rows.at[nb], agg_sh.at[didx.at[buf, j]], ssem).wait()
                pltpu.async_copy(
                    h_hbm.at[sidx.at[buf, j + NB - 1]], rows.at[nb], gsem)
            pltpu.async_copy(rows.at[b], agg_sh.at[didx.at[buf, j]], ssem, add=True)
            if with_cnt:
                pltpu.async_copy(ones, cnt_sh.at[didx.at[buf, j]], csem, add=True)

        for k in range(NB):
            pltpu.make_async_copy(
                rows.at[k], agg_sh.at[didx.at[buf, 0]], ssem).wait()
        if with_cnt:
            for k in range(CH):
                pltpu.make_async_copy(
                    ones, cnt_sh.at[didx.at[buf, 0]], csem).wait()

        @pl.when(ci + 1 < nch)
        def _():
            pltpu.make_async_copy(
                src_hbm.at[pl.ds(base_next, CH)], sidx.at[nbuf], isem).wait()
            pltpu.make_async_copy(
                dst_hbm.at[pl.ds(base_next, CH)], didx.at[nbuf], isem).wait()
        return carry

    lax.fori_loop(0, nch, chunk, 0)
    plsc.subcore_barrier()

    sl = pl.ds(s * RPT, RPT)

    @pl.when(c == 0)
    def _():
        pltpu.sync_copy(agg_sh.at[sl], agg0.at[sl])
        if with_cnt:
            pltpu.sync_copy(cnt_sh.at[sl], cnt0.at[sl])

    @pl.when(c == 1)
    def _():
        pltpu.sync_copy(agg_sh.at[sl], agg1.at[sl])
        if with_cnt:
            pltpu.sync_copy(cnt_sh.at[sl], cnt1.at[sl])


_sc_agg_cnt = pl.kernel(
    functools.partial(_sc_agg_body, True),
    out_type=(
        jax.ShapeDtypeStruct((NPAD, D), jnp.float32),
        jax.ShapeDtypeStruct((NPAD, D), jnp.float32),
        jax.ShapeDtypeStruct((NPAD,), jnp.float32),
        jax.ShapeDtypeStruct((NPAD,), jnp.float32),
    ),
    mesh=_mesh,
    scratch_types=[
        pltpu.VMEM((2, CH, G), jnp.int32),
        pltpu.VMEM((2, CH, G), jnp.int32),
        pltpu.VMEM((NB, G, D), jnp.float32),
        pltpu.VMEM((G,), jnp.float32),
        pltpu.VMEM_SHARED((NPAD, D), jnp.float32),
        pltpu.VMEM_SHARED((NPAD,), jnp.float32),
        pltpu.SemaphoreType.DMA,
        pltpu.SemaphoreType.DMA,
        pltpu.SemaphoreType.DMA,
        pltpu.SemaphoreType.DMA,
    ],
)

_sc_agg = pl.kernel(
    functools.partial(_sc_agg_body, False),
    out_type=(
        jax.ShapeDtypeStruct((NPAD, D), jnp.float32),
        jax.ShapeDtypeStruct((NPAD, D), jnp.float32),
    ),
    mesh=_mesh,
    scratch_types=[
        pltpu.VMEM((2, CH, G), jnp.int32),
        pltpu.VMEM((2, CH, G), jnp.int32),
        pltpu.VMEM((NB, G, D), jnp.float32),
        pltpu.VMEM_SHARED((NPAD, D), jnp.float32),
        pltpu.SemaphoreType.DMA,
        pltpu.SemaphoreType.DMA,
        pltpu.SemaphoreType.DMA,
    ],
)


BLK = 2048


def _tc_layer_body(act, h_ref, a0_ref, a1_ref, c0_ref, c1_ref,
                   wl_ref, bl_ref, wr_ref, o_ref):
    cnt = c0_ref[...] + c1_ref[...]
    inv = 1.0 / jnp.clip(cnt, 1.0, None)
    mean = (a0_ref[...].astype(jnp.float32)
            + a1_ref[...].astype(jnp.float32)) * inv
    z = (jnp.dot(mean, wl_ref[...], preferred_element_type=jnp.float32)
         + bl_ref[...]
         + jnp.dot(h_ref[...], wr_ref[...], preferred_element_type=jnp.float32))
    if act == "relu":
        z = jnp.maximum(z, 0.0)
    else:
        m = jnp.max(z, axis=-1, keepdims=True)
        e = jnp.exp(z - m)
        z = z - m - jnp.log(jnp.sum(e, axis=-1, keepdims=True))
    o_ref[...] = z


def _tc_layer(h, a0, a1, c0, c1, wlT, bl, wrT, act):
    row_spec = pl.BlockSpec((BLK, D), lambda i: (i, 0))
    cnt_spec = pl.BlockSpec((BLK, 1), lambda i: (i, 0))
    full = pl.BlockSpec((D, D), lambda i: (0, 0))
    bspec = pl.BlockSpec((1, D), lambda i: (0, 0))
    return pl.pallas_call(
        functools.partial(_tc_layer_body, act),
        grid=(NPAD // BLK,),
        in_specs=[row_spec, row_spec, row_spec, cnt_spec, cnt_spec,
                  full, bspec, full],
        out_specs=row_spec,
        out_shape=jax.ShapeDtypeStruct((NPAD, D), jnp.float32),
    )(h, a0, a1, c0, c1, wlT, bl, wrT)


def kernel(x, edge_index, Wl1, bl1, Wr1, Wl2, bl2, Wr2, Wl3, bl3, Wr3):
    ei = edge_index.astype(jnp.int32)
    src = jnp.concatenate(
        [ei[0], jnp.zeros((EPAD - N_EDGES,), jnp.int32)]).reshape(GROUPS_TOT, G)
    dst = jnp.concatenate(
        [ei[1], jnp.full((EPAD - N_EDGES,), N_NODES, jnp.int32)]).reshape(GROUPS_TOT, G)
    h0 = jnp.pad(x, ((0, NPAD - N_NODES), (0, 0)))
    z2d = jnp.zeros((NPAD, D), jnp.float32)
    z1d = jnp.zeros((NPAD,), jnp.float32)

    a0, a1, c0, c1 = _sc_agg_cnt(h0, src, dst, z2d, z1d)
    c0 = c0[:, None]
    c1 = c1[:, None]
    h1 = _tc_layer(h0, a0, a1, c0, c1, Wl1.T, bl1[None], Wr1.T, "relu")
    a0, a1 = _sc_agg(h1, src, dst, z2d)
    h2 = _tc_layer(h1, a0, a1, c0, c1, Wl2.T, bl2[None], Wr2.T, "relu")
    a0, a1 = _sc_agg(h2, src, dst, z2d)
    out = _tc_layer(h2, a0, a1, c0, c1, Wl3.T, bl3[None], Wr3.T, "logsoftmax")
    return out[:N_NODES]


# speedup vs baseline: 1.2071x; 1.0074x over previous
import functools

import jax
import jax.numpy as jnp
from jax import lax
from jax.experimental import pallas as pl
from jax.experimental.pallas import tpu as pltpu
from jax.experimental.pallas import tpu_sc as plsc

N_NODES = 10000
N_EDGES = 320000
D = 128

NC = 2
NS = 16
NW = NC * NS

NPAD = 10240
G = 64
EPAD = 327680
GROUPS_TOT = EPAD // G
GPT = GROUPS_TOT // NW
NB = 4
RPT = NPAD // NS

_mesh = plsc.VectorSubcoreMesh(core_axis_name="c", subcore_axis_name="s")


CH = 16
GPT0 = 304
GPT1 = 16


def _sc_agg_body(with_cnt, *refs):
    if with_cnt:
        (h_hbm, src_hbm, dst_hbm, z2d, z1d,
         agg0, agg1, cnt0, cnt1,
         sidx, didx, rows, ones, agg_sh, cnt_sh, gsem, ssem, csem, isem) = refs
    else:
        (h_hbm, src_hbm, dst_hbm, z2d,
         agg0, agg1,
         sidx, didx, rows, agg_sh, gsem, ssem, isem) = refs

    c = lax.axis_index("c")
    s = lax.axis_index("s")
    base = jnp.where(c == 0, s * GPT0, NS * GPT0 + s * GPT1)
    nch = jnp.where(c == 0, GPT0 // CH, GPT1 // CH)

    pltpu.sync_copy(z2d.at[pl.ds(s * RPT, RPT)], agg_sh.at[pl.ds(s * RPT, RPT)])
    if with_cnt:
        pltpu.sync_copy(z1d.at[pl.ds(s * RPT, RPT)], cnt_sh.at[pl.ds(s * RPT, RPT)])
        for k in range(G // 16):
            ones[pl.ds(k * 16, 16)] = jnp.full((16,), 1.0, jnp.float32)

    pltpu.sync_copy(src_hbm.at[pl.ds(base, CH)], sidx.at[0])
    pltpu.sync_copy(dst_hbm.at[pl.ds(base, CH)], didx.at[0])
    plsc.subcore_barrier()

    def chunk(ci, carry):
        buf = lax.rem(ci, 2)
        nbuf = lax.rem(ci + 1, 2)
        base_next = base + (ci + 1) * CH

        @pl.when(ci + 1 < nch)
        def _():
            pltpu.async_copy(src_hbm.at[pl.ds(base_next, CH)], sidx.at[nbuf], isem)
            pltpu.async_copy(dst_hbm.at[pl.ds(base_next, CH)], didx.at[nbuf], isem)

        for p in range(NB - 1):
            pltpu.async_copy(h_hbm.at[sidx.at[buf, p]], rows.at[p], gsem)
        for j in range(CH):
            b = j % NB
            pltpu.make_async_copy(
                h_hbm.at[sidx.at[buf, j]], rows.at[b], gsem).wait()
            if j + NB - 1 < CH:
                nb = (j + NB - 1) % NB
                if j >= 1:
                    pltpu.make_async_copy(
                        rows.at[nb], agg_sh.at[didx.at[buf, j]], ssem).wait()
                pltpu.async_copy(
                    h_hbm.at[sidx.at[buf, j + NB - 1]], rows.at[nb], gsem)
            pltpu.async_copy(rows.at[b], agg_sh.at[didx.at[buf, j]], ssem, add=True)
            if with_cnt:
                pltpu.async_copy(ones, cnt_sh.at[didx.at[buf, j]], csem, add=True)

        for k in range(NB):
            pltpu.make_async_copy(
                rows.at[k], agg_sh.at[didx.at[buf, 0]], ssem).wait()
        if with_cnt:
            for k in range(CH):
                pltpu.make_async_copy(
                    ones, cnt_sh.at[didx.at[buf, 0]], csem).wait()

        @pl.when(ci + 1 < nch)
        def _():
            pltpu.make_async_copy(
                src_hbm.at[pl.ds(base_next, CH)], sidx.at[nbuf], isem).wait()
            pltpu.make_async_copy(
                dst_hbm.at[pl.ds(base_next, CH)], didx.at[nbuf], isem).wait()
        return carry

    lax.fori_loop(0, nch, chunk, 0)
    plsc.subcore_barrier()

    sl = pl.ds(s * RPT, RPT)

    @pl.when(c == 0)
    def _():
        pltpu.sync_copy(agg_sh.at[sl], agg0.at[sl])
        if with_cnt:
            pltpu.sync_copy(cnt_sh.at[sl], cnt0.at[sl])

    @pl.when(c == 1)
    def _():
        pltpu.sync_copy(agg_sh.at[sl], agg1.at[sl])
        if with_cnt:
            pltpu.sync_copy(cnt_sh.at[sl], cnt1.at[sl])


_sc_agg_cnt = pl.kernel(
    functools.partial(_sc_agg_body, True),
    out_type=(
        jax.ShapeDtypeStruct((NPAD, D), jnp.float32),
        jax.ShapeDtypeStruct((NPAD, D), jnp.float32),
        jax.ShapeDtypeStruct((NPAD,), jnp.float32),
        jax.ShapeDtypeStruct((NPAD,), jnp.float32),
    ),
    mesh=_mesh,
    scratch_types=[
        pltpu.VMEM((2, CH, G), jnp.int32),
        pltpu.VMEM((2, CH, G), jnp.int32),
        pltpu.VMEM((NB, G, D), jnp.float32),
        pltpu.VMEM((G,), jnp.float32),
        pltpu.VMEM_SHARED((NPAD, D), jnp.float32),
        pltpu.VMEM_SHARED((NPAD,), jnp.float32),
        pltpu.SemaphoreType.DMA,
        pltpu.SemaphoreType.DMA,
        pltpu.SemaphoreType.DMA,
        pltpu.SemaphoreType.DMA,
    ],
)

_sc_agg = pl.kernel(
    functools.partial(_sc_agg_body, False),
    out_type=(
        jax.ShapeDtypeStruct((NPAD, D), jnp.float32),
        jax.ShapeDtypeStruct((NPAD, D), jnp.float32),
    ),
    mesh=_mesh,
    scratch_types=[
        pltpu.VMEM((2, CH, G), jnp.int32),
        pltpu.VMEM((2, CH, G), jnp.int32),
        pltpu.VMEM((NB, G, D), jnp.float32),
        pltpu.VMEM_SHARED((NPAD, D), jnp.float32),
        pltpu.SemaphoreType.DMA,
        pltpu.SemaphoreType.DMA,
        pltpu.SemaphoreType.DMA,
    ],
)


BLK = 2048


def _tc_layer_body(act, h_ref, a0_ref, a1_ref, c0_ref, c1_ref,
                   wl_ref, bl_ref, wr_ref, o_ref):
    cnt = c0_ref[...] + c1_ref[...]
    inv = 1.0 / jnp.clip(cnt, 1.0, None)
    mean = (a0_ref[...].astype(jnp.float32)
            + a1_ref[...].astype(jnp.float32)) * inv
    z = (jnp.dot(mean, wl_ref[...], preferred_element_type=jnp.float32)
         + bl_ref[...]
         + jnp.dot(h_ref[...], wr_ref[...], preferred_element_type=jnp.float32))
    if act == "relu":
        z = jnp.maximum(z, 0.0)
    else:
        m = jnp.max(z, axis=-1, keepdims=True)
        e = jnp.exp(z - m)
        z = z - m - jnp.log(jnp.sum(e, axis=-1, keepdims=True))
    o_ref[...] = z


def _tc_layer(h, a0, a1, c0, c1, wlT, bl, wrT, act):
    row_spec = pl.BlockSpec((BLK, D), lambda i: (i, 0))
    cnt_spec = pl.BlockSpec((BLK, 1), lambda i: (i, 0))
    full = pl.BlockSpec((D, D), lambda i: (0, 0))
    bspec = pl.BlockSpec((1, D), lambda i: (0, 0))
    return pl.pallas_call(
        functools.partial(_tc_layer_body, act),
        grid=(NPAD // BLK,),
        in_specs=[row_spec, row_spec, row_spec, cnt_spec, cnt_spec,
                  full, bspec, full],
        out_specs=row_spec,
        out_shape=jax.ShapeDtypeStruct((NPAD, D), jnp.float32),
    )(h, a0, a1, c0, c1, wlT, bl, wrT)


def kernel(x, edge_index, Wl1, bl1, Wr1, Wl2, bl2, Wr2, Wl3, bl3, Wr3):
    ei = edge_index.astype(jnp.int32)
    src = jnp.concatenate(
        [ei[0], jnp.zeros((EPAD - N_EDGES,), jnp.int32)]).reshape(GROUPS_TOT, G)
    dst = jnp.concatenate(
        [ei[1], jnp.full((EPAD - N_EDGES,), N_NODES, jnp.int32)]).reshape(GROUPS_TOT, G)
    h0 = jnp.pad(x, ((0, NPAD - N_NODES), (0, 0)))
    z2d = jnp.zeros((NPAD, D), jnp.float32)
    z1d = jnp.zeros((NPAD,), jnp.float32)

    a0, a1, c0, c1 = _sc_agg_cnt(h0, src, dst, z2d, z1d)
    c0 = c0[:, None]
    c1 = c1[:, None]
    h1 = _tc_layer(h0, a0, a1, c0, c1, Wl1.T, bl1[None], Wr1.T, "relu")
    a0, a1 = _sc_agg(h1, src, dst, z2d)
    h2 = _tc_layer(h1, a0, a1, c0, c1, Wl2.T, bl2[None], Wr2.T, "relu")
    a0, a1 = _sc_agg(h2, src, dst, z2d)
    out = _tc_layer(h2, a0, a1, c0, c1, Wl3.T, bl3[None], Wr3.T, "logsoftmax")
    return out[:N_NODES]
